# Initial kernel scaffold; baseline (speedup 1.0000x reference)
import jax
import jax.numpy as jnp
from jax.experimental import pallas as pl


def kernel(x, er_embed, pm_embed, token_table, pos_table, gamma, beta):
    raise NotImplementedError("write your pallas kernel here")


# speedup vs baseline: 2.6249x; 2.6249x over previous
import functools

import jax
import jax.numpy as jnp
from jax import lax
from jax.experimental import pallas as pl
from jax.experimental.pallas import tpu as pltpu
from jax.experimental.pallas import tpu_sc as plsc

_EPS = 1e-6


def _gather_sc(x_flat, token_table):
    N = x_flat.shape[0]
    V, D = token_table.shape
    info = plsc.get_sparse_core_info()
    NC, NS = info.num_cores, info.num_subcores
    NW = NC * NS
    K = 128
    per_w = N // NW
    n_chunks = per_w // K
    mesh = plsc.VectorSubcoreMesh(core_axis_name="c", subcore_axis_name="s")

    @functools.partial(
        pl.kernel,
        mesh=mesh,
        out_type=jax.ShapeDtypeStruct((N, D), jnp.float32),
        scratch_types=[
            pltpu.VMEM((K,), jnp.int32),
            pltpu.VMEM((K, D), jnp.float32),
            pltpu.SemaphoreType.DMA,
        ],
    )
    def k(x_hbm, tbl_hbm, out_hbm, idx_v, rows_v, sem):
        wid = lax.axis_index("s") * NC + lax.axis_index("c")
        base = wid * per_w

        def chunk(c, carry):
            off = base + c * K
            pltpu.sync_copy(x_hbm.at[pl.ds(off, K)], idx_v)
            pltpu.async_copy(tbl_hbm.at[idx_v], rows_v, sem).wait()
            pltpu.sync_copy(rows_v, out_hbm.at[pl.ds(off, K)])
            return carry

        lax.fori_loop(0, n_chunks, chunk, 0)

    return k(x_flat, token_table)


def _ln(h, g, b):
    mean = jnp.mean(h, axis=-1, keepdims=True)
    d = h - mean
    var = jnp.mean(d * d, axis=-1, keepdims=True)
    return g * (d * lax.rsqrt(var + _EPS)) + b


def _ln_all_tc(gathered, er_embed, pm_embed, gamma, beta, B):
    N, D = gathered.shape
    R = er_embed.shape[0]
    NB = 16
    TBLK = N // NB
    RBLK = R // NB

    def body(tok_ref, er_ref, pm_ref, g_ref, b_ref,
             tok_out_ref, er_out_ref, pm_out_ref):
        g = g_ref[0]
        b = b_ref[0]
        tok_out_ref[...] = _ln(tok_ref[...], g, b)
        for src, dst in ((er_ref, er_out_ref), (pm_ref, pm_out_ref)):
            y = _ln(src[...], g, b)
            dst[...] = jnp.broadcast_to(y[None], (B, RBLK, D))

    vec_spec = pl.BlockSpec((1, D), lambda i: (0, 0))
    row_spec_t = pl.BlockSpec((TBLK, D), lambda i: (i, 0))
    row_spec_r = pl.BlockSpec((RBLK, D), lambda i: (i, 0))
    out_spec_r = pl.BlockSpec((B, RBLK, D), lambda i: (0, i, 0))
    return pl.pallas_call(
        body,
        grid=(NB,),
        in_specs=[row_spec_t, row_spec_r, row_spec_r, vec_spec, vec_spec],
        out_specs=[row_spec_t, out_spec_r, out_spec_r],
        out_shape=[
            jax.ShapeDtypeStruct((N, D), jnp.float32),
            jax.ShapeDtypeStruct((B, R, D), jnp.float32),
            jax.ShapeDtypeStruct((B, R, D), jnp.float32),
        ],
    )(gathered, er_embed, pm_embed, gamma.reshape(1, D), beta.reshape(1, D))


def kernel(x, er_embed, pm_embed, token_table, pos_table, gamma, beta):
    B, S = x.shape
    D = token_table.shape[1]
    gathered = _gather_sc(x.reshape(-1), token_table)
    token_flat, er4, pm4 = _ln_all_tc(gathered, er_embed, pm_embed, gamma, beta, B)
    return token_flat.reshape(B, S, D), er4, pm4


# speedup vs baseline: 2.9269x; 1.1150x over previous
import functools

import jax
import jax.numpy as jnp
from jax import lax
from jax.experimental import pallas as pl
from jax.experimental.pallas import tpu as pltpu
from jax.experimental.pallas import tpu_sc as plsc

_EPS = 1e-6


def _gather_sc(x_flat, token_table):
    N = x_flat.shape[0]
    V, D = token_table.shape
    info = plsc.get_sparse_core_info()
    NC, NS = info.num_cores, info.num_subcores
    NW = NC * NS
    K = 128
    per_w = N // NW
    n_chunks = per_w // K
    mesh = plsc.VectorSubcoreMesh(core_axis_name="c", subcore_axis_name="s")

    @functools.partial(
        pl.kernel,
        mesh=mesh,
        out_type=jax.ShapeDtypeStruct((N, D), jnp.float32),
        scratch_types=[
            pltpu.VMEM((n_chunks, K), jnp.int32),
            pltpu.VMEM((2, K, D), jnp.float32),
            pltpu.SemaphoreType.DMA,
            pltpu.SemaphoreType.DMA,
            pltpu.SemaphoreType.DMA,
            pltpu.SemaphoreType.DMA,
        ],
    )
    def k(x_hbm, tbl_hbm, out_hbm, idx_v, rows_v, g0, g1, s0, s1):
        wid = lax.axis_index("s") * NC + lax.axis_index("c")
        base = wid * per_w
        gsem = (g0, g1)
        ssem = (s0, s1)
        pltpu.sync_copy(x_hbm.at[pl.ds(wid * n_chunks, n_chunks)], idx_v)
        hg = [None] * n_chunks
        hs = [None] * n_chunks
        hg[0] = pltpu.async_copy(tbl_hbm.at[idx_v.at[0]], rows_v.at[0], gsem[0])
        for c in range(n_chunks):
            buf = c % 2
            if c >= 1:
                hs[c - 1].wait()
            if c + 1 < n_chunks:
                nb = (c + 1) % 2
                hg[c + 1] = pltpu.async_copy(
                    tbl_hbm.at[idx_v.at[c + 1]], rows_v.at[nb], gsem[nb])
            hg[c].wait()
            hs[c] = pltpu.async_copy(
                rows_v.at[buf], out_hbm.at[pl.ds(base + c * K, K)], ssem[buf])
        hs[n_chunks - 1].wait()

    return k(x_flat.reshape(N // K, K), token_table)


def _ln(h, g, b):
    mean = jnp.mean(h, axis=-1, keepdims=True)
    d = h - mean
    var = jnp.mean(d * d, axis=-1, keepdims=True)
    return g * (d * lax.rsqrt(var + _EPS)) + b


def _ln_all_tc(gathered, er_embed, pm_embed, gamma, beta, B):
    N, D = gathered.shape
    R = er_embed.shape[0]
    NB = 16
    TBLK = N // NB
    RBLK = R // NB

    def body(tok_ref, er_ref, pm_ref, g_ref, b_ref,
             tok_out_ref, er_out_ref, pm_out_ref):
        g = g_ref[0]
        b = b_ref[0]
        tok_out_ref[...] = _ln(tok_ref[...], g, b)
        for src, dst in ((er_ref, er_out_ref), (pm_ref, pm_out_ref)):
            y = _ln(src[...], g, b)
            dst[...] = jnp.broadcast_to(y[None], (B, RBLK, D))

    vec_spec = pl.BlockSpec((1, D), lambda i: (0, 0))
    row_spec_t = pl.BlockSpec((TBLK, D), lambda i: (i, 0))
    row_spec_r = pl.BlockSpec((RBLK, D), lambda i: (i, 0))
    out_spec_r = pl.BlockSpec((B, RBLK, D), lambda i: (0, i, 0))
    return pl.pallas_call(
        body,
        grid=(NB,),
        in_specs=[row_spec_t, row_spec_r, row_spec_r, vec_spec, vec_spec],
        out_specs=[row_spec_t, out_spec_r, out_spec_r],
        out_shape=[
            jax.ShapeDtypeStruct((N, D), jnp.float32),
            jax.ShapeDtypeStruct((B, R, D), jnp.float32),
            jax.ShapeDtypeStruct((B, R, D), jnp.float32),
        ],
    )(gathered, er_embed, pm_embed, gamma.reshape(1, D), beta.reshape(1, D))


def kernel(x, er_embed, pm_embed, token_table, pos_table, gamma, beta):
    B, S = x.shape
    D = token_table.shape[1]
    gathered = _gather_sc(x.reshape(-1), token_table)
    token_flat, er4, pm4 = _ln_all_tc(gathered, er_embed, pm_embed, gamma, beta, B)
    return token_flat.reshape(B, S, D), er4, pm4
